# Initial kernel scaffold; baseline (speedup 1.0000x reference)
import jax
import jax.numpy as jnp
from jax.experimental import pallas as pl


def kernel(memory, source_nodes, timestamps, neighbors, edge_idxs, edge_times, edge_features, time_w, time_b, W1, b1, W2, b2):
    raise NotImplementedError("write your pallas kernel here")


# speedup vs baseline: 6.7221x; 6.7221x over previous
import functools

import jax
import jax.numpy as jnp
from jax import lax
from jax.experimental import pallas as pl
from jax.experimental.pallas import tpu as pltpu
from jax.experimental.pallas import tpu_sc as plsc


_CH = 128


def _sc_gather_build(n_mem, d_mem, n_edge, d_edge, kb, b):
    info = plsc.get_sparse_core_info()
    nw = info.num_cores * info.num_subcores
    rows_g = kb // nw
    rows_s = b // nw
    assert kb % (nw * _CH) == 0 and b % (nw * _CH) == 0

    mesh = plsc.VectorSubcoreMesh(core_axis_name="c", subcore_axis_name="s")

    @functools.partial(
        pl.kernel,
        out_type=(
            jax.ShapeDtypeStruct((kb, d_mem), jnp.float32),
            jax.ShapeDtypeStruct((kb, d_edge), jnp.float32),
            jax.ShapeDtypeStruct((b, d_mem), jnp.float32),
        ),
        mesh=mesh,
        compiler_params=pltpu.CompilerParams(use_tc_tiling_on_sc=False),
        scratch_types=[
            pltpu.VMEM((_CH,), jnp.int32),
            pltpu.VMEM((_CH,), jnp.int32),
            pltpu.VMEM((_CH, d_mem), jnp.float32),
            pltpu.VMEM((_CH, d_edge), jnp.float32),
            pltpu.SemaphoreType.DMA,
            pltpu.SemaphoreType.DMA,
        ],
    )
    def sc_gather(mem_hbm, ef_hbm, nbr_hbm, eidx_hbm, src_hbm,
                  g_hbm, e_hbm, s_hbm,
                  idx_v, idx2_v, rows_v, erows_v, sem, sem2):
        wid = lax.axis_index("s") * info.num_cores + lax.axis_index("c")

        gbase = wid * rows_g

        def gchunk(c, _):
            start = gbase + c * _CH
            pltpu.sync_copy(nbr_hbm.at[pl.ds(start, _CH)], idx_v)
            pltpu.sync_copy(eidx_hbm.at[pl.ds(start, _CH)], idx2_v)
            cp1 = pltpu.make_async_copy(mem_hbm.at[idx_v], rows_v, sem)
            cp2 = pltpu.make_async_copy(ef_hbm.at[idx2_v], erows_v, sem2)
            cp1.start()
            cp2.start()
            cp1.wait()
            cp2.wait()
            pltpu.sync_copy(rows_v, g_hbm.at[pl.ds(start, _CH)])
            pltpu.sync_copy(erows_v, e_hbm.at[pl.ds(start, _CH)])
            return _

        lax.fori_loop(0, rows_g // _CH, gchunk, 0)

        sbase = wid * rows_s

        def schunk(c, _):
            start = sbase + c * _CH
            pltpu.sync_copy(src_hbm.at[pl.ds(start, _CH)], idx_v)
            cp = pltpu.make_async_copy(mem_hbm.at[idx_v], rows_v, sem)
            cp.start()
            cp.wait()
            pltpu.sync_copy(rows_v, s_hbm.at[pl.ds(start, _CH)])
            return _

        lax.fori_loop(0, rows_s // _CH, schunk, 0)

    return sc_gather


_SB = 256

_INV2PI = 0.15915494309189535
_COS_C = (0.999999999919301, -19.739208758190394, 64.93939011212122,
          -85.45668534688727, 60.24246425821691, -26.406758112630982,
          7.806598832245124, -1.460935766960412)


def _cos_turns(t):
    n = jnp.floor(t + 0.5)
    u = t - n
    v = u * u
    acc = jnp.float32(_COS_C[-1])
    for c in _COS_C[-2::-1]:
        acc = acc * v + jnp.float32(c)
    return acc


def _tc_body(k, sb,
             g_ref, ef_ref, src_ref, ts_ref, et_ref, nbr_ref,
             w1a_ref, w1b_ref, w1c_ref, w2a_ref, w2b_ref, w2c_ref,
             tw_ref, tb_ref, b1_ref, b2_ref, out_ref):
    f32 = jnp.float32
    dflat = (ts_ref[...] - et_ref[...]).reshape(k * sb, 1)
    tw_t = tw_ref[...] * _INV2PI
    tb_t = tb_ref[...] * _INV2PI
    t_feat = _cos_turns(dflat * tw_t + tb_t)

    g = g_ref[...].reshape(k * sb, g_ref.shape[-1])
    ef = ef_ref[...].reshape(k * sb, ef_ref.shape[-1])

    acc = jnp.dot(g, w1a_ref[...], preferred_element_type=f32)
    acc += jnp.dot(t_feat, w1b_ref[...], preferred_element_type=f32)
    acc += jnp.dot(ef, w1c_ref[...], preferred_element_type=f32)
    acc += b1_ref[...]
    h = jnp.maximum(acc, 0.0)
    h = jnp.where(nbr_ref[...].reshape(k * sb, 1) == 0, 0.0, h)
    hsum = h.reshape(k, sb, h.shape[-1]).sum(axis=0)

    src_time = _cos_turns(tb_t)
    const = jnp.dot(src_time, w2c_ref[...], preferred_element_type=f32)

    out = jnp.dot(hsum, w2a_ref[...], preferred_element_type=f32)
    out += jnp.dot(src_ref[...], w2b_ref[...], preferred_element_type=f32)
    out += const + b2_ref[...]
    out_ref[...] = jnp.maximum(out, 0.0)


def _tc_fused(g3, ef3, src, ts_col, et_col, nbr_col,
              w1a, w1b, w1c, w2a, w2b, w2c, tw, tb, b1, b2):
    k, b, d_mem = g3.shape
    d_edge = ef3.shape[-1]
    d_time = tw.shape[-1]
    d_emb = w1a.shape[-1]
    grid = (b // _SB,)

    bmap3 = lambda i: (0, i, 0)
    colspec = pl.BlockSpec((k, _SB, 1), bmap3)
    full2 = lambda r, c: pl.BlockSpec((r, c), lambda i: (0, 0))
    return pl.pallas_call(
        functools.partial(_tc_body, k, _SB),
        grid=grid,
        in_specs=[
            pl.BlockSpec((k, _SB, d_mem), bmap3),
            pl.BlockSpec((k, _SB, d_edge), bmap3),
            pl.BlockSpec((_SB, d_mem), lambda i: (i, 0)),
            colspec,
            colspec,
            colspec,
            full2(d_mem, d_emb),
            full2(d_time, d_emb),
            full2(d_edge, d_emb),
            full2(d_emb, d_emb),
            full2(d_mem, d_emb),
            full2(d_time, d_emb),
            full2(1, d_time),
            full2(1, d_time),
            full2(1, d_emb),
            full2(1, d_emb),
        ],
        out_specs=pl.BlockSpec((_SB, d_emb), lambda i: (i, 0)),
        out_shape=jax.ShapeDtypeStruct((b, d_emb), jnp.float32),
    )(g3, ef3, src, ts_col, et_col, nbr_col,
      w1a, w1b, w1c, w2a, w2b, w2c, tw, tb, b1, b2)


def kernel(memory, source_nodes, timestamps, neighbors, edge_idxs, edge_times,
           edge_features, time_w, time_b, W1, b1, W2, b2):
    b, k = neighbors.shape
    n_mem, d_mem = memory.shape
    n_edge, d_edge = edge_features.shape
    d_time = time_w.shape[0]
    d_emb = W1.shape[1]

    nbr_t = neighbors.T.astype(jnp.int32)
    eidx_t = edge_idxs.T.astype(jnp.int32)
    et_t = edge_times.T

    sc_gather = _sc_gather_build(n_mem, d_mem, n_edge, d_edge, k * b, b)
    g_flat, e_flat, src_rows = sc_gather(
        memory, edge_features,
        nbr_t.reshape(-1), eidx_t.reshape(-1), source_nodes.astype(jnp.int32))

    w1a = W1[:d_mem]
    w1b = W1[d_mem:d_mem + d_time]
    w1c = W1[d_mem + d_time:]
    w2a = W2[:d_emb]
    w2b = W2[d_emb:d_emb + d_mem]
    w2c = W2[d_emb + d_mem:]

    out = _tc_fused(
        g_flat.reshape(k, b, d_mem),
        e_flat.reshape(k, b, d_edge),
        src_rows,
        jnp.broadcast_to(timestamps[None, :, None], (k, b, 1)),
        et_t.reshape(k, b, 1),
        nbr_t.reshape(k, b, 1),
        w1a, w1b, w1c, w2a, w2b, w2c,
        time_w.reshape(1, d_time), time_b.reshape(1, d_time),
        b1.reshape(1, d_emb), b2.reshape(1, d_emb))
    return out
